# baseline (speedup 1.0000x reference)
import functools

import jax
import jax.numpy as jnp
from jax import lax
from jax.experimental import pallas as pl
from jax.experimental.pallas import tpu as pltpu
from jax.experimental.pallas import tpu_sc as plsc

_N = 10000
_E = 320000
_D = 128
_NB = 20
_CUT = 5.0
_NW = 32
_LANES = 16

_mesh = plsc.VectorSubcoreMesh(core_axis_name="c", subcore_axis_name="s")

_CHUNK = 128
_FULL = _N // _CHUNK
_TAIL = _N - _FULL * _CHUNK


@functools.partial(
    pl.kernel,
    mesh=_mesh,
    out_type=jax.ShapeDtypeStruct((_N, _D), jnp.float32),
    scratch_types=[
        pltpu.VMEM((_CHUNK,), jnp.int32),
        pltpu.VMEM((_CHUNK, _D), jnp.float32),
        pltpu.VMEM((_TAIL,), jnp.int32),
        pltpu.VMEM((_TAIL, _D), jnp.float32),
        pltpu.SemaphoreType.DMA,
    ],
)
def _emb_gather(at_no_hbm, table_hbm, out_hbm, idx_v, rows_v, idxt_v, rowst_v, sem):
    wid = lax.axis_index("s") * 2 + lax.axis_index("c")
    for k in range((_FULL + _NW - 1) // _NW):
        c = wid + k * _NW

        @pl.when(c < _FULL)
        def _():
            base = c * _CHUNK
            pltpu.sync_copy(at_no_hbm.at[pl.ds(base, _CHUNK)], idx_v)
            pltpu.async_copy(table_hbm.at[idx_v], rows_v, sem).wait()
            pltpu.sync_copy(rows_v, out_hbm.at[pl.ds(base, _CHUNK)])

    @pl.when(wid == _NW - 1)
    def _():
        base = _FULL * _CHUNK
        pltpu.sync_copy(at_no_hbm.at[pl.ds(base, _TAIL)], idxt_v)
        pltpu.async_copy(table_hbm.at[idxt_v], rowst_v, sem).wait()
        pltpu.sync_copy(rowst_v, out_hbm.at[pl.ds(base, _TAIL)])


_CE = _E // _NW


@functools.partial(
    pl.kernel,
    mesh=_mesh,
    out_type=(
        jax.ShapeDtypeStruct((_E,), jnp.float32),
        jax.ShapeDtypeStruct((_E,), jnp.float32),
        jax.ShapeDtypeStruct((_E,), jnp.float32),
    ),
    scratch_types=[
        pltpu.VMEM((3 * _N,), jnp.float32),
        pltpu.VMEM((_CE,), jnp.int32),
        pltpu.VMEM((_CE,), jnp.int32),
        pltpu.VMEM((_CE,), jnp.float32),
        pltpu.VMEM((_CE,), jnp.float32),
        pltpu.VMEM((_CE,), jnp.float32),
    ],
    compiler_params=pltpu.CompilerParams(needs_layout_passes=False),
)
def _edge_vec(src_hbm, dst_hbm, pos_hbm, vx_hbm, vy_hbm, vz_hbm,
              pos_v, src_v, dst_v, vx_v, vy_v, vz_v):
    wid = lax.axis_index("s") * 2 + lax.axis_index("c")
    base = wid * _CE
    pltpu.sync_copy(pos_hbm, pos_v)
    pltpu.sync_copy(src_hbm.at[pl.ds(base, _CE)], src_v)
    pltpu.sync_copy(dst_hbm.at[pl.ds(base, _CE)], dst_v)

    def body(i, carry):
        off = i * _LANES
        s = src_v[pl.ds(off, _LANES)] * 3
        d = dst_v[pl.ds(off, _LANES)] * 3
        sx = plsc.load_gather(pos_v, [s])
        dx = plsc.load_gather(pos_v, [d])
        sy = plsc.load_gather(pos_v, [s + 1])
        dy = plsc.load_gather(pos_v, [d + 1])
        sz = plsc.load_gather(pos_v, [s + 2])
        dz = plsc.load_gather(pos_v, [d + 2])
        vx_v[pl.ds(off, _LANES)] = dx - sx
        vy_v[pl.ds(off, _LANES)] = dy - sy
        vz_v[pl.ds(off, _LANES)] = dz - sz
        return carry

    lax.fori_loop(0, _CE // _LANES, body, 0)
    pltpu.sync_copy(vx_v, vx_hbm.at[pl.ds(base, _CE)])
    pltpu.sync_copy(vy_v, vy_hbm.at[pl.ds(base, _CE)])
    pltpu.sync_copy(vz_v, vz_hbm.at[pl.ds(base, _CE)])


_ROWS = _E // 128
_B = 25
_G = _ROWS // _B

_SIN_C = (9.9999999948e-01, -1.6666666108e-01, 8.3333236832e-03,
          -1.9840647444e-04, 2.7538255745e-06, -2.4752145009e-08,
          1.3697371161e-10)
_COS_C = (9.9999999992e-01, -4.9999999889e-01, 4.1666664158e-02,
          -1.3888867464e-03, 2.4800691215e-05, -2.7536989152e-07,
          2.0620727662e-09, -9.7749972032e-12)
_PI = 3.14159265358979323846


def _tc_body(vx_ref, vy_ref, vz_ref, rbf_ref, fcut_ref, rsh_ref):
    vx = vx_ref[0]
    vy = vy_ref[0]
    vz = vz_ref[0]
    d2 = vx * vx + vy * vy + vz * vz
    d = jnp.sqrt(d2)
    inv = 1.0 / d
    theta = d * (_PI / _CUT)
    q = jnp.round(theta * (0.5 / _PI))
    t = theta - q * (2.0 * _PI)
    t2 = t * t
    sp = jnp.float32(_SIN_C[-1])
    for c in _SIN_C[-2::-1]:
        sp = sp * t2 + c
    s1 = sp * t
    cq = jnp.float32(_COS_C[-1])
    for c in _COS_C[-2::-1]:
        cq = cq * t2 + c
    c1 = cq

    scale_inv = jnp.sqrt(2.0 / _CUT) * inv
    two_c = 2.0 * c1
    cols = []
    s_prev = jnp.zeros_like(s1)
    s_cur = s1
    for _ in range(_NB):
        cols.append(scale_inv * s_cur)
        s_prev, s_cur = s_cur, two_c * s_cur - s_prev
    s_stack = jnp.stack(cols, axis=0)
    eye_nb = (lax.broadcasted_iota(jnp.int32, (_NB, _NB), 0)
              == lax.broadcasted_iota(jnp.int32, (_NB, _NB), 1)
              ).astype(jnp.float32)
    rbf_ref[...] = lax.dot_general(
        s_stack, eye_nb, (((0,), (0,)), ((), ())),
        preferred_element_type=jnp.float32).reshape(_B * 128, _NB)
    fcut = 0.5 * (c1 + 1.0) * (d < _CUT).astype(jnp.float32)
    one1 = jnp.ones((1, 1), jnp.float32)
    fcut_ref[...] = lax.dot_general(
        fcut[None], one1, (((0,), (0,)), ((), ())),
        preferred_element_type=jnp.float32).reshape(_B * 128, 1)
    r_stack = jnp.stack([vx * inv, vy * inv, vz * inv], axis=0)
    eye3 = (lax.broadcasted_iota(jnp.int32, (3, 3), 0)
            == lax.broadcasted_iota(jnp.int32, (3, 3), 1)).astype(jnp.float32)
    rsh_ref[...] = lax.dot_general(
        r_stack, eye3, (((0,), (0,)), ((), ())),
        preferred_element_type=jnp.float32).reshape(_B * 128, 3)


_tc_call = pl.pallas_call(
    _tc_body,
    grid=(_G,),
    in_specs=[pl.BlockSpec((1, _B, 128), lambda i: (i, 0, 0))] * 3,
    out_specs=[
        pl.BlockSpec((_B * 128, _NB), lambda i: (i, 0)),
        pl.BlockSpec((_B * 128, 1), lambda i: (i, 0)),
        pl.BlockSpec((_B * 128, 3), lambda i: (i, 0)),
    ],
    out_shape=[
        jax.ShapeDtypeStruct((_E, _NB), jnp.float32),
        jax.ShapeDtypeStruct((_E, 1), jnp.float32),
        jax.ShapeDtypeStruct((_E, 3), jnp.float32),
    ],
)


def kernel(at_no, pos, edge_index, emb_table):
    at_no = at_no.astype(jnp.int32)
    src = edge_index[0].astype(jnp.int32)
    dst = edge_index[1].astype(jnp.int32)
    posf = pos.reshape(-1)
    x_scalar = _emb_gather(at_no, emb_table)
    vx, vy, vz = _edge_vec(src, dst, posf)
    shp = (_G, _B, 128)
    rbf, fcut, rsh = _tc_call(vx.reshape(shp), vy.reshape(shp),
                              vz.reshape(shp))
    return (x_scalar, rbf, fcut, rsh)


# speedup vs baseline: 1.3557x; 1.3557x over previous
import functools

import jax
import jax.numpy as jnp
from jax import lax
from jax.experimental import pallas as pl
from jax.experimental.pallas import tpu as pltpu
from jax.experimental.pallas import tpu_sc as plsc

_N = 10000
_E = 320000
_D = 128
_NB = 20
_CUT = 5.0
_NW = 32
_LANES = 16

_mesh = plsc.VectorSubcoreMesh(core_axis_name="c", subcore_axis_name="s")

_CHUNK = 128
_FULL = _N // _CHUNK
_TAIL = _N - _FULL * _CHUNK


@functools.partial(
    pl.kernel,
    mesh=_mesh,
    out_type=jax.ShapeDtypeStruct((_N, _D), jnp.float32),
    scratch_types=[
        pltpu.VMEM((_CHUNK,), jnp.int32),
        pltpu.VMEM((_CHUNK, _D), jnp.float32),
        pltpu.VMEM((_TAIL,), jnp.int32),
        pltpu.VMEM((_TAIL, _D), jnp.float32),
        pltpu.SemaphoreType.DMA,
    ],
)
def _emb_gather(at_no_hbm, table_hbm, out_hbm, idx_v, rows_v, idxt_v, rowst_v, sem):
    wid = lax.axis_index("s") * 2 + lax.axis_index("c")
    for k in range((_FULL + _NW - 1) // _NW):
        c = wid + k * _NW

        @pl.when(c < _FULL)
        def _():
            base = c * _CHUNK
            pltpu.sync_copy(at_no_hbm.at[pl.ds(base, _CHUNK)], idx_v)
            pltpu.async_copy(table_hbm.at[idx_v], rows_v, sem).wait()
            pltpu.sync_copy(rows_v, out_hbm.at[pl.ds(base, _CHUNK)])

    @pl.when(wid == _NW - 1)
    def _():
        base = _FULL * _CHUNK
        pltpu.sync_copy(at_no_hbm.at[pl.ds(base, _TAIL)], idxt_v)
        pltpu.async_copy(table_hbm.at[idxt_v], rowst_v, sem).wait()
        pltpu.sync_copy(rowst_v, out_hbm.at[pl.ds(base, _TAIL)])


_CE = _E // _NW


@functools.partial(
    pl.kernel,
    mesh=_mesh,
    out_type=(
        jax.ShapeDtypeStruct((_E,), jnp.float32),
        jax.ShapeDtypeStruct((_E,), jnp.float32),
        jax.ShapeDtypeStruct((_E,), jnp.float32),
    ),
    scratch_types=[
        pltpu.VMEM((3 * _N,), jnp.float32),
        pltpu.VMEM((_CE,), jnp.int32),
        pltpu.VMEM((_CE,), jnp.int32),
        pltpu.VMEM((_CE,), jnp.float32),
        pltpu.VMEM((_CE,), jnp.float32),
        pltpu.VMEM((_CE,), jnp.float32),
    ],
    compiler_params=pltpu.CompilerParams(needs_layout_passes=False),
)
def _edge_vec(src_hbm, dst_hbm, pos_hbm, vx_hbm, vy_hbm, vz_hbm,
              pos_v, src_v, dst_v, vx_v, vy_v, vz_v):
    wid = lax.axis_index("s") * 2 + lax.axis_index("c")
    base = wid * _CE
    pltpu.sync_copy(pos_hbm, pos_v)
    pltpu.sync_copy(src_hbm.at[pl.ds(base, _CE)], src_v)
    pltpu.sync_copy(dst_hbm.at[pl.ds(base, _CE)], dst_v)

    def body(i, carry):
        off = i * _LANES
        s = src_v[pl.ds(off, _LANES)] * 3
        d = dst_v[pl.ds(off, _LANES)] * 3
        sx = plsc.load_gather(pos_v, [s])
        dx = plsc.load_gather(pos_v, [d])
        sy = plsc.load_gather(pos_v, [s + 1])
        dy = plsc.load_gather(pos_v, [d + 1])
        sz = plsc.load_gather(pos_v, [s + 2])
        dz = plsc.load_gather(pos_v, [d + 2])
        vx_v[pl.ds(off, _LANES)] = dx - sx
        vy_v[pl.ds(off, _LANES)] = dy - sy
        vz_v[pl.ds(off, _LANES)] = dz - sz
        return carry

    lax.fori_loop(0, _CE // _LANES, body, 0)
    pltpu.sync_copy(vx_v, vx_hbm.at[pl.ds(base, _CE)])
    pltpu.sync_copy(vy_v, vy_hbm.at[pl.ds(base, _CE)])
    pltpu.sync_copy(vz_v, vz_hbm.at[pl.ds(base, _CE)])


_ROWS = _E // 128
_B = 25
_G = _ROWS // _B

_SIN_C = (9.9999999948e-01, -1.6666666108e-01, 8.3333236832e-03,
          -1.9840647444e-04, 2.7538255745e-06, -2.4752145009e-08,
          1.3697371161e-10)
_COS_C = (9.9999999992e-01, -4.9999999889e-01, 4.1666664158e-02,
          -1.3888867464e-03, 2.4800691215e-05, -2.7536989152e-07,
          2.0620727662e-09, -9.7749972032e-12)
_PI = 3.14159265358979323846


def _sin_poly(t):
    t2 = t * t
    sp = jnp.float32(_SIN_C[-1])
    for c in _SIN_C[-2::-1]:
        sp = sp * t2 + c
    return sp * t


def _cos_poly(t):
    t2 = t * t
    cq = jnp.float32(_COS_C[-1])
    for c in _COS_C[-2::-1]:
        cq = cq * t2 + c
    return cq


def _reduce(x):
    q = jnp.floor(x * (0.5 / _PI) + 0.5)
    return x - q * (2.0 * _PI)


_BE = 512


def _tc_body(vx_ref, vy_ref, vz_ref, rbf_ref, fcut_ref, rsh_ref):
    vx = vx_ref[...]
    vy = vy_ref[...]
    vz = vz_ref[...]
    d2 = vx * vx + vy * vy + vz * vz
    d = jnp.sqrt(d2)
    inv = 1.0 / d
    theta = d * (_PI / _CUT)
    nvals = (lax.broadcasted_iota(jnp.int32, (_NB, 1), 0) + 1
             ).astype(jnp.float32)
    args = nvals * theta[None, :]
    scale_inv = (jnp.sqrt(2.0 / _CUT) * inv)[None, :]
    rbf_ref[...] = _sin_poly(_reduce(args)) * scale_inv
    c1 = _cos_poly(_reduce(theta))
    fcut_ref[...] = (0.5 * (c1 + 1.0)
                     * (d < _CUT).astype(jnp.float32))[None, :]
    rsh_ref[...] = jnp.concatenate(
        [(vx * inv)[None, :], (vy * inv)[None, :], (vz * inv)[None, :]],
        axis=0)


_tc_call = pl.pallas_call(
    _tc_body,
    grid=(_E // _BE,),
    in_specs=[pl.BlockSpec((_BE,), lambda i: (i,))] * 3,
    out_specs=[
        pl.BlockSpec((_NB, _BE), lambda i: (0, i)),
        pl.BlockSpec((1, _BE), lambda i: (0, i)),
        pl.BlockSpec((3, _BE), lambda i: (0, i)),
    ],
    out_shape=[
        jax.ShapeDtypeStruct((_NB, _E), jnp.float32),
        jax.ShapeDtypeStruct((1, _E), jnp.float32),
        jax.ShapeDtypeStruct((3, _E), jnp.float32),
    ],
)


def kernel(at_no, pos, edge_index, emb_table):
    at_no = at_no.astype(jnp.int32)
    src = edge_index[0].astype(jnp.int32)
    dst = edge_index[1].astype(jnp.int32)
    posf = pos.reshape(-1)
    x_scalar = _emb_gather(at_no, emb_table)
    vx, vy, vz = _edge_vec(src, dst, posf)
    rbf_t, fcut_t, rsh_t = _tc_call(vx, vy, vz)
    return (x_scalar, rbf_t.T, fcut_t.T, rsh_t.T)


# speedup vs baseline: 4.1733x; 3.0783x over previous
import functools

import jax
import jax.numpy as jnp
from jax import lax
from jax.experimental import pallas as pl
from jax.experimental.pallas import tpu as pltpu
from jax.experimental.pallas import tpu_sc as plsc

_N = 10000
_E = 320000
_D = 128
_NB = 20
_CUT = 5.0
_NW = 32
_LANES = 16

_mesh = plsc.VectorSubcoreMesh(core_axis_name="c", subcore_axis_name="s")

_CHUNK = 128
_FULL = _N // _CHUNK
_TAIL = _N - _FULL * _CHUNK


@functools.partial(
    pl.kernel,
    mesh=_mesh,
    out_type=jax.ShapeDtypeStruct((_N, _D), jnp.float32),
    scratch_types=[
        pltpu.VMEM((_CHUNK,), jnp.int32),
        pltpu.VMEM((_CHUNK, _D), jnp.float32),
        pltpu.VMEM((_TAIL,), jnp.int32),
        pltpu.VMEM((_TAIL, _D), jnp.float32),
        pltpu.SemaphoreType.DMA,
    ],
)
def _emb_gather(at_no_hbm, table_hbm, out_hbm, idx_v, rows_v, idxt_v, rowst_v, sem):
    wid = lax.axis_index("s") * 2 + lax.axis_index("c")
    for k in range((_FULL + _NW - 1) // _NW):
        c = wid + k * _NW

        @pl.when(c < _FULL)
        def _():
            base = c * _CHUNK
            pltpu.sync_copy(at_no_hbm.at[pl.ds(base, _CHUNK)], idx_v)
            pltpu.async_copy(table_hbm.at[idx_v], rows_v, sem).wait()
            pltpu.sync_copy(rows_v, out_hbm.at[pl.ds(base, _CHUNK)])

    @pl.when(wid == _NW - 1)
    def _():
        base = _FULL * _CHUNK
        pltpu.sync_copy(at_no_hbm.at[pl.ds(base, _TAIL)], idxt_v)
        pltpu.async_copy(table_hbm.at[idxt_v], rowst_v, sem).wait()
        pltpu.sync_copy(rowst_v, out_hbm.at[pl.ds(base, _TAIL)])


_CE = _E // _NW


@functools.partial(
    pl.kernel,
    mesh=_mesh,
    out_type=(
        jax.ShapeDtypeStruct((_E,), jnp.float32),
        jax.ShapeDtypeStruct((_E,), jnp.float32),
        jax.ShapeDtypeStruct((_E,), jnp.float32),
    ),
    scratch_types=[
        pltpu.VMEM((3 * _N,), jnp.float32),
        pltpu.VMEM((_CE,), jnp.int32),
        pltpu.VMEM((_CE,), jnp.int32),
        pltpu.VMEM((_CE,), jnp.float32),
        pltpu.VMEM((_CE,), jnp.float32),
        pltpu.VMEM((_CE,), jnp.float32),
    ],
    compiler_params=pltpu.CompilerParams(needs_layout_passes=False),
)
def _edge_vec(src_hbm, dst_hbm, pos_hbm, vx_hbm, vy_hbm, vz_hbm,
              pos_v, src_v, dst_v, vx_v, vy_v, vz_v):
    wid = lax.axis_index("s") * 2 + lax.axis_index("c")
    base = wid * _CE
    pltpu.sync_copy(pos_hbm, pos_v)
    pltpu.sync_copy(src_hbm.at[pl.ds(base, _CE)], src_v)
    pltpu.sync_copy(dst_hbm.at[pl.ds(base, _CE)], dst_v)

    def body(i, carry):
        off = i * _LANES
        s = src_v[pl.ds(off, _LANES)] * 3
        d = dst_v[pl.ds(off, _LANES)] * 3
        sx = plsc.load_gather(pos_v, [s])
        dx = plsc.load_gather(pos_v, [d])
        sy = plsc.load_gather(pos_v, [s + 1])
        dy = plsc.load_gather(pos_v, [d + 1])
        sz = plsc.load_gather(pos_v, [s + 2])
        dz = plsc.load_gather(pos_v, [d + 2])
        vx_v[pl.ds(off, _LANES)] = dx - sx
        vy_v[pl.ds(off, _LANES)] = dy - sy
        vz_v[pl.ds(off, _LANES)] = dz - sz
        return carry

    lax.fori_loop(0, _CE // _LANES, body, 0)
    pltpu.sync_copy(vx_v, vx_hbm.at[pl.ds(base, _CE)])
    pltpu.sync_copy(vy_v, vy_hbm.at[pl.ds(base, _CE)])
    pltpu.sync_copy(vz_v, vz_hbm.at[pl.ds(base, _CE)])


_ROWS = _E // 128
_B = 25
_G = _ROWS // _B

_SIN_C = (9.9999999948e-01, -1.6666666108e-01, 8.3333236832e-03,
          -1.9840647444e-04, 2.7538255745e-06, -2.4752145009e-08,
          1.3697371161e-10)
_COS_C = (9.9999999992e-01, -4.9999999889e-01, 4.1666664158e-02,
          -1.3888867464e-03, 2.4800691215e-05, -2.7536989152e-07,
          2.0620727662e-09, -9.7749972032e-12)
_PI = 3.14159265358979323846


def _sin_poly(t):
    t2 = t * t
    sp = jnp.float32(_SIN_C[-1])
    for c in _SIN_C[-2::-1]:
        sp = sp * t2 + c
    return sp * t


def _cos_poly(t):
    t2 = t * t
    cq = jnp.float32(_COS_C[-1])
    for c in _COS_C[-2::-1]:
        cq = cq * t2 + c
    return cq


def _reduce(x):
    q = jnp.floor(x * (0.5 / _PI) + 0.5)
    return x - q * (2.0 * _PI)


_BE = 6400


def _tc_body(vx_ref, vy_ref, vz_ref, rbf_ref, fcut_ref, rsh_ref):
    vx = vx_ref[...]
    vy = vy_ref[...]
    vz = vz_ref[...]
    d2 = vx * vx + vy * vy + vz * vz
    d = jnp.sqrt(d2)
    inv = 1.0 / d
    theta = d * (_PI / _CUT)
    nvals = (lax.broadcasted_iota(jnp.int32, (_NB, 1), 0) + 1
             ).astype(jnp.float32)
    args = nvals * theta
    scale_inv = jnp.sqrt(2.0 / _CUT) * inv
    rbf_ref[...] = _sin_poly(_reduce(args)) * scale_inv
    c1 = _cos_poly(_reduce(theta))
    fcut_ref[...] = 0.5 * (c1 + 1.0) * (d < _CUT).astype(jnp.float32)
    rsh_ref[...] = jnp.concatenate(
        [vx * inv, vy * inv, vz * inv], axis=0)


_tc_call = pl.pallas_call(
    _tc_body,
    grid=(_E // _BE,),
    in_specs=[pl.BlockSpec((1, _BE), lambda i: (0, i))] * 3,
    out_specs=[
        pl.BlockSpec((_NB, _BE), lambda i: (0, i)),
        pl.BlockSpec((1, _BE), lambda i: (0, i)),
        pl.BlockSpec((3, _BE), lambda i: (0, i)),
    ],
    out_shape=[
        jax.ShapeDtypeStruct((_NB, _E), jnp.float32),
        jax.ShapeDtypeStruct((1, _E), jnp.float32),
        jax.ShapeDtypeStruct((3, _E), jnp.float32),
    ],
)


def kernel(at_no, pos, edge_index, emb_table):
    at_no = at_no.astype(jnp.int32)
    src = edge_index[0].astype(jnp.int32)
    dst = edge_index[1].astype(jnp.int32)
    posf = pos.reshape(-1)
    x_scalar = _emb_gather(at_no, emb_table)
    vx, vy, vz = _edge_vec(src, dst, posf)
    rbf_t, fcut_t, rsh_t = _tc_call(vx.reshape(1, _E), vy.reshape(1, _E),
                                    vz.reshape(1, _E))
    return (x_scalar, rbf_t.T, fcut_t.T, rsh_t.T)


# speedup vs baseline: 4.7093x; 1.1284x over previous
import functools

import jax
import jax.numpy as jnp
from jax import lax
from jax.experimental import pallas as pl
from jax.experimental.pallas import tpu as pltpu
from jax.experimental.pallas import tpu_sc as plsc

_N = 10000
_E = 320000
_D = 128
_NB = 20
_CUT = 5.0
_NW = 32
_LANES = 16

_mesh = plsc.VectorSubcoreMesh(core_axis_name="c", subcore_axis_name="s")

_CHUNK = 128
_FULL = _N // _CHUNK
_TAIL = _N - _FULL * _CHUNK


@functools.partial(
    pl.kernel,
    mesh=_mesh,
    out_type=jax.ShapeDtypeStruct((_N, _D), jnp.float32),
    scratch_types=[
        pltpu.VMEM((_CHUNK,), jnp.int32),
        pltpu.VMEM((_CHUNK, _D), jnp.float32),
        pltpu.VMEM((_TAIL,), jnp.int32),
        pltpu.VMEM((_TAIL, _D), jnp.float32),
        pltpu.SemaphoreType.DMA,
    ],
)
def _emb_gather(at_no_hbm, table_hbm, out_hbm, idx_v, rows_v, idxt_v, rowst_v, sem):
    wid = lax.axis_index("s") * 2 + lax.axis_index("c")
    for k in range((_FULL + _NW - 1) // _NW):
        c = wid + k * _NW

        @pl.when(c < _FULL)
        def _():
            base = c * _CHUNK
            pltpu.sync_copy(at_no_hbm.at[pl.ds(base, _CHUNK)], idx_v)
            pltpu.async_copy(table_hbm.at[idx_v], rows_v, sem).wait()
            pltpu.sync_copy(rows_v, out_hbm.at[pl.ds(base, _CHUNK)])

    @pl.when(wid == _NW - 1)
    def _():
        base = _FULL * _CHUNK
        pltpu.sync_copy(at_no_hbm.at[pl.ds(base, _TAIL)], idxt_v)
        pltpu.async_copy(table_hbm.at[idxt_v], rowst_v, sem).wait()
        pltpu.sync_copy(rowst_v, out_hbm.at[pl.ds(base, _TAIL)])


_CE = _E // _NW


@functools.partial(
    pl.kernel,
    mesh=_mesh,
    out_type=(
        jax.ShapeDtypeStruct((_E,), jnp.float32),
        jax.ShapeDtypeStruct((_E,), jnp.float32),
        jax.ShapeDtypeStruct((_E,), jnp.float32),
    ),
    scratch_types=[
        pltpu.VMEM((3 * _N,), jnp.float32),
        pltpu.VMEM((_CE,), jnp.int32),
        pltpu.VMEM((_CE,), jnp.int32),
        pltpu.VMEM((_CE,), jnp.float32),
        pltpu.VMEM((_CE,), jnp.float32),
        pltpu.VMEM((_CE,), jnp.float32),
    ],
    compiler_params=pltpu.CompilerParams(needs_layout_passes=False),
)
def _edge_vec(src_hbm, dst_hbm, pos_hbm, vx_hbm, vy_hbm, vz_hbm,
              pos_v, src_v, dst_v, vx_v, vy_v, vz_v):
    wid = lax.axis_index("s") * 2 + lax.axis_index("c")
    base = wid * _CE
    pltpu.sync_copy(pos_hbm, pos_v)
    pltpu.sync_copy(src_hbm.at[pl.ds(base, _CE)], src_v)
    pltpu.sync_copy(dst_hbm.at[pl.ds(base, _CE)], dst_v)

    @plsc.parallel_loop(0, _CE // _LANES, unroll=8)
    def _loop(i):
        off = i * _LANES
        s = src_v[pl.ds(off, _LANES)] * 3
        d = dst_v[pl.ds(off, _LANES)] * 3
        sx = plsc.load_gather(pos_v, [s])
        dx = plsc.load_gather(pos_v, [d])
        sy = plsc.load_gather(pos_v, [s + 1])
        dy = plsc.load_gather(pos_v, [d + 1])
        sz = plsc.load_gather(pos_v, [s + 2])
        dz = plsc.load_gather(pos_v, [d + 2])
        vx_v[pl.ds(off, _LANES)] = dx - sx
        vy_v[pl.ds(off, _LANES)] = dy - sy
        vz_v[pl.ds(off, _LANES)] = dz - sz
    pltpu.sync_copy(vx_v, vx_hbm.at[pl.ds(base, _CE)])
    pltpu.sync_copy(vy_v, vy_hbm.at[pl.ds(base, _CE)])
    pltpu.sync_copy(vz_v, vz_hbm.at[pl.ds(base, _CE)])


_ROWS = _E // 128
_B = 25
_G = _ROWS // _B

_SIN_C = (9.9999999948e-01, -1.6666666108e-01, 8.3333236832e-03,
          -1.9840647444e-04, 2.7538255745e-06, -2.4752145009e-08,
          1.3697371161e-10)
_COS_C = (9.9999999992e-01, -4.9999999889e-01, 4.1666664158e-02,
          -1.3888867464e-03, 2.4800691215e-05, -2.7536989152e-07,
          2.0620727662e-09, -9.7749972032e-12)
_PI = 3.14159265358979323846


def _sin_poly(t):
    t2 = t * t
    sp = jnp.float32(_SIN_C[-1])
    for c in _SIN_C[-2::-1]:
        sp = sp * t2 + c
    return sp * t


def _cos_poly(t):
    t2 = t * t
    cq = jnp.float32(_COS_C[-1])
    for c in _COS_C[-2::-1]:
        cq = cq * t2 + c
    return cq


def _reduce(x):
    q = jnp.floor(x * (0.5 / _PI) + 0.5)
    return x - q * (2.0 * _PI)


_BE = 12800


def _tc_body(vx_ref, vy_ref, vz_ref, rbf_ref, fcut_ref, rsh_ref):
    vx = vx_ref[...]
    vy = vy_ref[...]
    vz = vz_ref[...]
    d2 = vx * vx + vy * vy + vz * vz
    d = jnp.sqrt(d2)
    inv = 1.0 / d
    theta = d * (_PI / _CUT)
    nvals = (lax.broadcasted_iota(jnp.int32, (_NB, 1), 0) + 1
             ).astype(jnp.float32)
    args = nvals * theta
    scale_inv = jnp.sqrt(2.0 / _CUT) * inv
    rbf_ref[...] = _sin_poly(_reduce(args)) * scale_inv
    c1 = _cos_poly(_reduce(theta))
    fcut_ref[...] = 0.5 * (c1 + 1.0) * (d < _CUT).astype(jnp.float32)
    rsh_ref[...] = jnp.concatenate(
        [vx * inv, vy * inv, vz * inv], axis=0)


_tc_call = pl.pallas_call(
    _tc_body,
    grid=(_E // _BE,),
    in_specs=[pl.BlockSpec((1, _BE), lambda i: (0, i))] * 3,
    out_specs=[
        pl.BlockSpec((_NB, _BE), lambda i: (0, i)),
        pl.BlockSpec((1, _BE), lambda i: (0, i)),
        pl.BlockSpec((3, _BE), lambda i: (0, i)),
    ],
    out_shape=[
        jax.ShapeDtypeStruct((_NB, _E), jnp.float32),
        jax.ShapeDtypeStruct((1, _E), jnp.float32),
        jax.ShapeDtypeStruct((3, _E), jnp.float32),
    ],
)


def kernel(at_no, pos, edge_index, emb_table):
    at_no = at_no.astype(jnp.int32)
    src = edge_index[0].astype(jnp.int32)
    dst = edge_index[1].astype(jnp.int32)
    posf = pos.reshape(-1)
    x_scalar = _emb_gather(at_no, emb_table)
    vx, vy, vz = _edge_vec(src, dst, posf)
    rbf_t, fcut_t, rsh_t = _tc_call(vx.reshape(1, _E), vy.reshape(1, _E),
                                    vz.reshape(1, _E))
    return (x_scalar, rbf_t.T, fcut_t.T, rsh_t.T)


# speedup vs baseline: 5.2977x; 1.1249x over previous
import functools

import jax
import jax.numpy as jnp
from jax import lax
from jax.experimental import pallas as pl
from jax.experimental.pallas import tpu as pltpu
from jax.experimental.pallas import tpu_sc as plsc

_N = 10000
_E = 320000
_D = 128
_NB = 20
_CUT = 5.0
_NW = 32
_LANES = 16

_mesh = plsc.VectorSubcoreMesh(core_axis_name="c", subcore_axis_name="s")

_CHUNK = 128
_FULL = _N // _CHUNK
_TAIL = _N - _FULL * _CHUNK


@functools.partial(
    pl.kernel,
    mesh=_mesh,
    out_type=jax.ShapeDtypeStruct((_N, _D), jnp.float32),
    scratch_types=[
        pltpu.VMEM((_CHUNK,), jnp.int32),
        pltpu.VMEM((_CHUNK, _D), jnp.float32),
        pltpu.VMEM((_TAIL,), jnp.int32),
        pltpu.VMEM((_TAIL, _D), jnp.float32),
        pltpu.SemaphoreType.DMA,
    ],
)
def _emb_gather(at_no_hbm, table_hbm, out_hbm, idx_v, rows_v, idxt_v, rowst_v, sem):
    wid = lax.axis_index("s") * 2 + lax.axis_index("c")
    for k in range((_FULL + _NW - 1) // _NW):
        c = wid + k * _NW

        @pl.when(c < _FULL)
        def _():
            base = c * _CHUNK
            pltpu.sync_copy(at_no_hbm.at[pl.ds(base, _CHUNK)], idx_v)
            pltpu.async_copy(table_hbm.at[idx_v], rows_v, sem).wait()
            pltpu.sync_copy(rows_v, out_hbm.at[pl.ds(base, _CHUNK)])

    @pl.when(wid == _NW - 1)
    def _():
        base = _FULL * _CHUNK
        pltpu.sync_copy(at_no_hbm.at[pl.ds(base, _TAIL)], idxt_v)
        pltpu.async_copy(table_hbm.at[idxt_v], rowst_v, sem).wait()
        pltpu.sync_copy(rowst_v, out_hbm.at[pl.ds(base, _TAIL)])


_CE = 9984
_NTILES = _E // 128
_EXTRA = _NTILES - (_CE // 128) * _NW


@functools.partial(
    pl.kernel,
    mesh=_mesh,
    out_type=(
        jax.ShapeDtypeStruct((1, _E), jnp.float32),
        jax.ShapeDtypeStruct((1, _E), jnp.float32),
        jax.ShapeDtypeStruct((1, _E), jnp.float32),
    ),
    scratch_types=[
        pltpu.VMEM((3 * _N,), jnp.float32),
        pltpu.VMEM((_CE,), jnp.int32),
        pltpu.VMEM((_CE,), jnp.int32),
        pltpu.VMEM((_CE,), jnp.float32),
        pltpu.VMEM((_CE,), jnp.float32),
        pltpu.VMEM((_CE,), jnp.float32),
    ],
    compiler_params=pltpu.CompilerParams(needs_layout_passes=False),
)
def _edge_vec(src_hbm, dst_hbm, pos_hbm, vx_hbm, vy_hbm, vz_hbm,
              pos_v, src_v, dst_v, vx_v, vy_v, vz_v):
    wid = lax.axis_index("s") * 2 + lax.axis_index("c")
    pltpu.sync_copy(pos_hbm, pos_v)

    def gather_range(base, n_groups):
        pltpu.sync_copy(src_hbm.at[pl.ds(base, n_groups * _LANES)],
                        src_v.at[pl.ds(0, n_groups * _LANES)])
        pltpu.sync_copy(dst_hbm.at[pl.ds(base, n_groups * _LANES)],
                        dst_v.at[pl.ds(0, n_groups * _LANES)])

        @plsc.parallel_loop(0, n_groups, unroll=8)
        def _loop(i):
            off = i * _LANES
            s = src_v[pl.ds(off, _LANES)] * 3
            d = dst_v[pl.ds(off, _LANES)] * 3
            sx = plsc.load_gather(pos_v, [s])
            dx = plsc.load_gather(pos_v, [d])
            sy = plsc.load_gather(pos_v, [s + 1])
            dy = plsc.load_gather(pos_v, [d + 1])
            sz = plsc.load_gather(pos_v, [s + 2])
            dz = plsc.load_gather(pos_v, [d + 2])
            vx_v[pl.ds(off, _LANES)] = dx - sx
            vy_v[pl.ds(off, _LANES)] = dy - sy
            vz_v[pl.ds(off, _LANES)] = dz - sz

        n = n_groups * _LANES
        pltpu.sync_copy(vx_v.at[pl.ds(0, n)], vx_hbm.at[0, pl.ds(base, n)])
        pltpu.sync_copy(vy_v.at[pl.ds(0, n)], vy_hbm.at[0, pl.ds(base, n)])
        pltpu.sync_copy(vz_v.at[pl.ds(0, n)], vz_hbm.at[0, pl.ds(base, n)])

    gather_range(wid * _CE, _CE // _LANES)

    @pl.when(wid < _EXTRA)
    def _():
        gather_range(_NW * _CE + wid * 128, 128 // _LANES)


_ROWS = _E // 128
_B = 25
_G = _ROWS // _B

_SIN_C = (9.9999999948e-01, -1.6666666108e-01, 8.3333236832e-03,
          -1.9840647444e-04, 2.7538255745e-06, -2.4752145009e-08,
          1.3697371161e-10)
_COS_C = (9.9999999992e-01, -4.9999999889e-01, 4.1666664158e-02,
          -1.3888867464e-03, 2.4800691215e-05, -2.7536989152e-07,
          2.0620727662e-09, -9.7749972032e-12)
_PI = 3.14159265358979323846


def _sin_poly(t):
    t2 = t * t
    sp = jnp.float32(_SIN_C[-1])
    for c in _SIN_C[-2::-1]:
        sp = sp * t2 + c
    return sp * t


def _cos_poly(t):
    t2 = t * t
    cq = jnp.float32(_COS_C[-1])
    for c in _COS_C[-2::-1]:
        cq = cq * t2 + c
    return cq


def _reduce(x):
    q = jnp.floor(x * (0.5 / _PI) + 0.5)
    return x - q * (2.0 * _PI)


_BE = 12800


def _tc_body(vx_ref, vy_ref, vz_ref, rbf_ref, fcut_ref, rsh_ref):
    vx = vx_ref[...]
    vy = vy_ref[...]
    vz = vz_ref[...]
    d2 = vx * vx + vy * vy + vz * vz
    d = jnp.sqrt(d2)
    inv = 1.0 / d
    theta = d * (_PI / _CUT)
    nvals = (lax.broadcasted_iota(jnp.int32, (_NB, 1), 0) + 1
             ).astype(jnp.float32)
    args = nvals * theta
    scale_inv = jnp.sqrt(2.0 / _CUT) * inv
    rbf_ref[...] = _sin_poly(_reduce(args)) * scale_inv
    c1 = _cos_poly(_reduce(theta))
    fcut_ref[...] = 0.5 * (c1 + 1.0) * (d < _CUT).astype(jnp.float32)
    rsh_ref[...] = jnp.concatenate(
        [vx * inv, vy * inv, vz * inv], axis=0)


_tc_call = pl.pallas_call(
    _tc_body,
    grid=(_E // _BE,),
    in_specs=[pl.BlockSpec((1, _BE), lambda i: (0, i))] * 3,
    out_specs=[
        pl.BlockSpec((_NB, _BE), lambda i: (0, i)),
        pl.BlockSpec((1, _BE), lambda i: (0, i)),
        pl.BlockSpec((3, _BE), lambda i: (0, i)),
    ],
    out_shape=[
        jax.ShapeDtypeStruct((_NB, _E), jnp.float32),
        jax.ShapeDtypeStruct((1, _E), jnp.float32),
        jax.ShapeDtypeStruct((3, _E), jnp.float32),
    ],
)


def kernel(at_no, pos, edge_index, emb_table):
    at_no = at_no.astype(jnp.int32)
    src = edge_index[0].astype(jnp.int32)
    dst = edge_index[1].astype(jnp.int32)
    posf = pos.reshape(-1)
    x_scalar = _emb_gather(at_no, emb_table)
    vx, vy, vz = _edge_vec(src, dst, posf)
    rbf_t, fcut_t, rsh_t = _tc_call(vx, vy, vz)
    return (x_scalar, rbf_t.T, fcut_t.T, rsh_t.T)


# speedup vs baseline: 6.4062x; 1.2092x over previous
import functools

import jax
import jax.numpy as jnp
from jax import lax
from jax.experimental import pallas as pl
from jax.experimental.pallas import tpu as pltpu
from jax.experimental.pallas import tpu_sc as plsc

_N = 10000
_E = 320000
_D = 128
_NB = 20
_CUT = 5.0
_NW = 32
_LANES = 16

_mesh = plsc.VectorSubcoreMesh(core_axis_name="c", subcore_axis_name="s")

_CHUNK = 128
_FULL = _N // _CHUNK
_TAIL = _N - _FULL * _CHUNK


@functools.partial(
    pl.kernel,
    mesh=_mesh,
    out_type=jax.ShapeDtypeStruct((_N, _D), jnp.float32),
    scratch_types=[
        pltpu.VMEM((_CHUNK,), jnp.int32),
        pltpu.VMEM((_CHUNK, _D), jnp.float32),
        pltpu.VMEM((_TAIL,), jnp.int32),
        pltpu.VMEM((_TAIL, _D), jnp.float32),
        pltpu.SemaphoreType.DMA,
    ],
)
def _emb_gather(at_no_hbm, table_hbm, out_hbm, idx_v, rows_v, idxt_v, rowst_v, sem):
    wid = lax.axis_index("s") * 2 + lax.axis_index("c")
    for k in range((_FULL + _NW - 1) // _NW):
        c = wid + k * _NW

        @pl.when(c < _FULL)
        def _():
            base = c * _CHUNK
            pltpu.sync_copy(at_no_hbm.at[pl.ds(base, _CHUNK)], idx_v)
            pltpu.async_copy(table_hbm.at[idx_v], rows_v, sem).wait()
            pltpu.sync_copy(rows_v, out_hbm.at[pl.ds(base, _CHUNK)])

    @pl.when(wid == _NW - 1)
    def _():
        base = _FULL * _CHUNK
        pltpu.sync_copy(at_no_hbm.at[pl.ds(base, _TAIL)], idxt_v)
        pltpu.async_copy(table_hbm.at[idxt_v], rowst_v, sem).wait()
        pltpu.sync_copy(rowst_v, out_hbm.at[pl.ds(base, _TAIL)])


_CE = 9984
_NTILES = _E // 128
_EXTRA = _NTILES - (_CE // 128) * _NW


@functools.partial(
    pl.kernel,
    mesh=_mesh,
    out_type=(
        jax.ShapeDtypeStruct((1, _E), jnp.float32),
        jax.ShapeDtypeStruct((1, _E), jnp.float32),
        jax.ShapeDtypeStruct((1, _E), jnp.float32),
    ),
    scratch_types=[
        pltpu.VMEM((3 * _N,), jnp.float32),
        pltpu.VMEM((2, _CE), jnp.int32),
        pltpu.VMEM((_CE,), jnp.float32),
        pltpu.VMEM((_CE,), jnp.float32),
        pltpu.VMEM((_CE,), jnp.float32),
        pltpu.SemaphoreType.DMA,
    ],
    compiler_params=pltpu.CompilerParams(needs_layout_passes=False),
)
def _edge_vec(edge_hbm, pos_hbm, vx_hbm, vy_hbm, vz_hbm,
              pos_v, ed_v, vx_v, vy_v, vz_v, sem):
    wid = lax.axis_index("s") * 2 + lax.axis_index("c")
    cp_pos = pltpu.async_copy(pos_hbm, pos_v, sem)

    def gather_range(base, n_groups):
        n = n_groups * _LANES
        pltpu.sync_copy(edge_hbm.at[:, pl.ds(base, n)],
                        ed_v.at[:, pl.ds(0, n)])

        @plsc.parallel_loop(0, n_groups, unroll=8)
        def _loop(i):
            off = i * _LANES
            s = ed_v[0, pl.ds(off, _LANES)] * 3
            d = ed_v[1, pl.ds(off, _LANES)] * 3
            sx = plsc.load_gather(pos_v, [s])
            dx = plsc.load_gather(pos_v, [d])
            sy = plsc.load_gather(pos_v, [s + 1])
            dy = plsc.load_gather(pos_v, [d + 1])
            sz = plsc.load_gather(pos_v, [s + 2])
            dz = plsc.load_gather(pos_v, [d + 2])
            vx_v[pl.ds(off, _LANES)] = dx - sx
            vy_v[pl.ds(off, _LANES)] = dy - sy
            vz_v[pl.ds(off, _LANES)] = dz - sz

        cx = pltpu.async_copy(vx_v.at[pl.ds(0, n)],
                              vx_hbm.at[0, pl.ds(base, n)], sem)
        cy = pltpu.async_copy(vy_v.at[pl.ds(0, n)],
                              vy_hbm.at[0, pl.ds(base, n)], sem)
        cz = pltpu.async_copy(vz_v.at[pl.ds(0, n)],
                              vz_hbm.at[0, pl.ds(base, n)], sem)
        return cx, cy, cz

    cp_pos.wait()
    cs = gather_range(wid * _CE, _CE // _LANES)
    for c in cs:
        c.wait()

    @pl.when(wid < _EXTRA)
    def _():
        for c in gather_range(_NW * _CE + wid * 128, 128 // _LANES):
            c.wait()


_ROWS = _E // 128
_B = 25
_G = _ROWS // _B

_SIN_C = (9.9999999948e-01, -1.6666666108e-01, 8.3333236832e-03,
          -1.9840647444e-04, 2.7538255745e-06, -2.4752145009e-08,
          1.3697371161e-10)
_COS_C = (9.9999999992e-01, -4.9999999889e-01, 4.1666664158e-02,
          -1.3888867464e-03, 2.4800691215e-05, -2.7536989152e-07,
          2.0620727662e-09, -9.7749972032e-12)
_PI = 3.14159265358979323846


def _sin_poly(t):
    t2 = t * t
    sp = jnp.float32(_SIN_C[-1])
    for c in _SIN_C[-2::-1]:
        sp = sp * t2 + c
    return sp * t


def _cos_poly(t):
    t2 = t * t
    cq = jnp.float32(_COS_C[-1])
    for c in _COS_C[-2::-1]:
        cq = cq * t2 + c
    return cq


def _reduce(x):
    q = jnp.floor(x * (0.5 / _PI) + 0.5)
    return x - q * (2.0 * _PI)


_BE = 12800


def _tc_body(vx_ref, vy_ref, vz_ref, rbf_ref, fcut_ref, rsh_ref):
    vx = vx_ref[...]
    vy = vy_ref[...]
    vz = vz_ref[...]
    d2 = vx * vx + vy * vy + vz * vz
    d = jnp.sqrt(d2)
    inv = 1.0 / d
    theta = d * (_PI / _CUT)
    nvals = (lax.broadcasted_iota(jnp.int32, (_NB, 1), 0) + 1
             ).astype(jnp.float32)
    args = nvals * theta
    scale_inv = jnp.sqrt(2.0 / _CUT) * inv
    rbf_ref[...] = _sin_poly(_reduce(args)) * scale_inv
    c1 = _cos_poly(_reduce(theta))
    fcut_ref[...] = 0.5 * (c1 + 1.0) * (d < _CUT).astype(jnp.float32)
    rsh_ref[...] = jnp.concatenate(
        [vx * inv, vy * inv, vz * inv], axis=0)


_tc_call = pl.pallas_call(
    _tc_body,
    grid=(_E // _BE,),
    in_specs=[pl.BlockSpec((1, _BE), lambda i: (0, i))] * 3,
    out_specs=[
        pl.BlockSpec((_NB, _BE), lambda i: (0, i)),
        pl.BlockSpec((1, _BE), lambda i: (0, i)),
        pl.BlockSpec((3, _BE), lambda i: (0, i)),
    ],
    out_shape=[
        jax.ShapeDtypeStruct((_NB, _E), jnp.float32),
        jax.ShapeDtypeStruct((1, _E), jnp.float32),
        jax.ShapeDtypeStruct((3, _E), jnp.float32),
    ],
)


def kernel(at_no, pos, edge_index, emb_table):
    at_no = at_no.astype(jnp.int32)
    posf = pos.reshape(-1)
    x_scalar = _emb_gather(at_no, emb_table)
    vx, vy, vz = _edge_vec(edge_index.astype(jnp.int32), posf)
    rbf_t, fcut_t, rsh_t = _tc_call(vx, vy, vz)
    return (x_scalar, rbf_t.T, fcut_t.T, rsh_t.T)


# speedup vs baseline: 6.8485x; 1.0690x over previous
import functools

import jax
import jax.numpy as jnp
from jax import lax
from jax.experimental import pallas as pl
from jax.experimental.pallas import tpu as pltpu
from jax.experimental.pallas import tpu_sc as plsc

_N = 10000
_E = 320000
_D = 128
_NB = 20
_CUT = 5.0
_NW = 32
_LANES = 16

_mesh = plsc.VectorSubcoreMesh(core_axis_name="c", subcore_axis_name="s")

_CHUNK = 128
_FULL = _N // _CHUNK
_TAIL = _N - _FULL * _CHUNK


@functools.partial(
    pl.kernel,
    mesh=_mesh,
    out_type=jax.ShapeDtypeStruct((_N, _D), jnp.float32),
    scratch_types=[
        pltpu.VMEM((_CHUNK,), jnp.int32),
        pltpu.VMEM((_CHUNK, _D), jnp.float32),
        pltpu.VMEM((_TAIL,), jnp.int32),
        pltpu.VMEM((_TAIL, _D), jnp.float32),
        pltpu.SemaphoreType.DMA,
    ],
)
def _emb_gather(at_no_hbm, table_hbm, out_hbm, idx_v, rows_v, idxt_v, rowst_v, sem):
    wid = lax.axis_index("s") * 2 + lax.axis_index("c")
    for k in range((_FULL + _NW - 1) // _NW):
        c = wid + k * _NW

        @pl.when(c < _FULL)
        def _():
            base = c * _CHUNK
            pltpu.sync_copy(at_no_hbm.at[pl.ds(base, _CHUNK)], idx_v)
            pltpu.async_copy(table_hbm.at[idx_v], rows_v, sem).wait()
            pltpu.sync_copy(rows_v, out_hbm.at[pl.ds(base, _CHUNK)])

    @pl.when(wid == _NW - 1)
    def _():
        base = _FULL * _CHUNK
        pltpu.sync_copy(at_no_hbm.at[pl.ds(base, _TAIL)], idxt_v)
        pltpu.async_copy(table_hbm.at[idxt_v], rowst_v, sem).wait()
        pltpu.sync_copy(rowst_v, out_hbm.at[pl.ds(base, _TAIL)])


_CE = 9984
_NTILES = _E // 128
_EXTRA = _NTILES - (_CE // 128) * _NW


@functools.partial(
    pl.kernel,
    mesh=_mesh,
    out_type=(
        jax.ShapeDtypeStruct((1, _E), jnp.float32),
        jax.ShapeDtypeStruct((1, _E), jnp.float32),
        jax.ShapeDtypeStruct((1, _E), jnp.float32),
    ),
    scratch_types=[
        pltpu.VMEM((3 * _N,), jnp.float32),
        pltpu.VMEM((2, _CE), jnp.int32),
        pltpu.VMEM((_CE,), jnp.float32),
        pltpu.VMEM((_CE,), jnp.float32),
        pltpu.VMEM((_CE,), jnp.float32),
        pltpu.SemaphoreType.DMA,
    ],
    compiler_params=pltpu.CompilerParams(needs_layout_passes=False),
)
def _edge_vec(edge_hbm, pos_hbm, vx_hbm, vy_hbm, vz_hbm,
              pos_v, ed_v, vx_v, vy_v, vz_v, sem):
    wid = lax.axis_index("s") * 2 + lax.axis_index("c")
    cp_pos = pltpu.async_copy(pos_hbm, pos_v, sem)

    def gather_range(base, n_groups):
        n = n_groups * _LANES
        pltpu.sync_copy(edge_hbm.at[:, pl.ds(base, n)],
                        ed_v.at[:, pl.ds(0, n)])

        @plsc.parallel_loop(0, n_groups, unroll=8)
        def _loop(i):
            off = i * _LANES
            s = ed_v[0, pl.ds(off, _LANES)]
            d = ed_v[1, pl.ds(off, _LANES)]
            sx = plsc.load_gather(pos_v, [s])
            dx = plsc.load_gather(pos_v, [d])
            sy = plsc.load_gather(pos_v, [s + _N])
            dy = plsc.load_gather(pos_v, [d + _N])
            sz = plsc.load_gather(pos_v, [s + 2 * _N])
            dz = plsc.load_gather(pos_v, [d + 2 * _N])
            vx_v[pl.ds(off, _LANES)] = dx - sx
            vy_v[pl.ds(off, _LANES)] = dy - sy
            vz_v[pl.ds(off, _LANES)] = dz - sz

        cx = pltpu.async_copy(vx_v.at[pl.ds(0, n)],
                              vx_hbm.at[0, pl.ds(base, n)], sem)
        cy = pltpu.async_copy(vy_v.at[pl.ds(0, n)],
                              vy_hbm.at[0, pl.ds(base, n)], sem)
        cz = pltpu.async_copy(vz_v.at[pl.ds(0, n)],
                              vz_hbm.at[0, pl.ds(base, n)], sem)
        return cx, cy, cz

    cp_pos.wait()
    cs = gather_range(wid * _CE, _CE // _LANES)
    for c in cs:
        c.wait()

    @pl.when(wid < _EXTRA)
    def _():
        for c in gather_range(_NW * _CE + wid * 128, 128 // _LANES):
            c.wait()


_ROWS = _E // 128
_B = 25
_G = _ROWS // _B

_SIN_C = (9.9999999948e-01, -1.6666666108e-01, 8.3333236832e-03,
          -1.9840647444e-04, 2.7538255745e-06, -2.4752145009e-08,
          1.3697371161e-10)
_COS_C = (9.9999999992e-01, -4.9999999889e-01, 4.1666664158e-02,
          -1.3888867464e-03, 2.4800691215e-05, -2.7536989152e-07,
          2.0620727662e-09, -9.7749972032e-12)
_PI = 3.14159265358979323846


def _sin_poly(t):
    t2 = t * t
    sp = jnp.float32(_SIN_C[-1])
    for c in _SIN_C[-2::-1]:
        sp = sp * t2 + c
    return sp * t


def _cos_poly(t):
    t2 = t * t
    cq = jnp.float32(_COS_C[-1])
    for c in _COS_C[-2::-1]:
        cq = cq * t2 + c
    return cq


def _reduce(x):
    q = jnp.floor(x * (0.5 / _PI) + 0.5)
    return x - q * (2.0 * _PI)


_BE = 12800


def _tc_body(vx_ref, vy_ref, vz_ref, rbf_ref, fcut_ref, rsh_ref):
    vx = vx_ref[...]
    vy = vy_ref[...]
    vz = vz_ref[...]
    d2 = vx * vx + vy * vy + vz * vz
    d = jnp.sqrt(d2)
    inv = 1.0 / d
    theta = d * (_PI / _CUT)
    nvals = (lax.broadcasted_iota(jnp.int32, (_NB, 1), 0) + 1
             ).astype(jnp.float32)
    args = nvals * theta
    scale_inv = jnp.sqrt(2.0 / _CUT) * inv
    rbf_ref[...] = _sin_poly(_reduce(args)) * scale_inv
    c1 = _cos_poly(_reduce(theta))
    fcut_ref[...] = 0.5 * (c1 + 1.0) * (d < _CUT).astype(jnp.float32)
    rsh_ref[...] = jnp.concatenate(
        [vx * inv, vy * inv, vz * inv], axis=0)


_tc_call = pl.pallas_call(
    _tc_body,
    grid=(_E // _BE,),
    in_specs=[pl.BlockSpec((1, _BE), lambda i: (0, i))] * 3,
    out_specs=[
        pl.BlockSpec((_NB, _BE), lambda i: (0, i)),
        pl.BlockSpec((1, _BE), lambda i: (0, i)),
        pl.BlockSpec((3, _BE), lambda i: (0, i)),
    ],
    out_shape=[
        jax.ShapeDtypeStruct((_NB, _E), jnp.float32),
        jax.ShapeDtypeStruct((1, _E), jnp.float32),
        jax.ShapeDtypeStruct((3, _E), jnp.float32),
    ],
)


def kernel(at_no, pos, edge_index, emb_table):
    at_no = at_no.astype(jnp.int32)
    posf = pos.T.reshape(-1)
    x_scalar = _emb_gather(at_no, emb_table)
    vx, vy, vz = _edge_vec(edge_index.astype(jnp.int32), posf)
    rbf_t, fcut_t, rsh_t = _tc_call(vx, vy, vz)
    return (x_scalar, rbf_t.T, fcut_t.T, rsh_t.T)
